# baseline (speedup 1.0000x reference)
import jax
import jax.numpy as jnp
from jax import lax
from jax.experimental import pallas as pl

B, T, D = 64, 1024, 32
K = 1024
TB = 4096


def _vq_body(h_ref, w1_ref, w2_ref, q_ref, idx_ref):
    ha = h_ref[...]
    h = ha[:, :D]
    dist = lax.dot_general(ha, w1_ref[...], (((1,), (0,)), ((), ())),
                           preferred_element_type=jnp.float32)
    m = jnp.min(dist, axis=1, keepdims=True)
    onehot = (dist <= m).astype(jnp.float32)
    qi = lax.dot_general(onehot, w2_ref[...], (((1,), (0,)), ((), ())),
                         preferred_element_type=jnp.float32)
    q = qi[:, :D]
    q_ref[...] = h + (q - h)
    idx_ref[...] = qi[:, D:D + 1].astype(jnp.int32)


@jax.jit
def kernel(h, codebook):
    bsz, t, d = h.shape
    n = bsz * t
    grid = n // TB
    flat = jnp.concatenate(
        [h.reshape(n, d), jnp.ones((n, 1), jnp.float32)], axis=1)
    c2 = jnp.sum(codebook ** 2, axis=1)[None, :]
    w1 = jnp.concatenate([-2.0 * codebook.T, c2], axis=0)
    w2 = jnp.concatenate(
        [codebook, lax.broadcasted_iota(jnp.float32, (K, 1), 0)], axis=1)
    q_flat, idx_col = pl.pallas_call(
        _vq_body,
        grid=(grid,),
        in_specs=[
            pl.BlockSpec((TB, d + 1), lambda i: (i, 0)),
            pl.BlockSpec((d + 1, K), lambda i: (0, 0)),
            pl.BlockSpec((K, d + 1), lambda i: (0, 0)),
        ],
        out_specs=[
            pl.BlockSpec((TB, d), lambda i: (i, 0)),
            pl.BlockSpec((TB, 1), lambda i: (i, 0)),
        ],
        out_shape=[
            jax.ShapeDtypeStruct((n, d), jnp.float32),
            jax.ShapeDtypeStruct((n, 1), jnp.int32),
        ],
    )(flat, w1, w2)
    return q_flat.reshape(bsz, t, d), idx_col.reshape(bsz, t)


# speedup vs baseline: 1.0920x; 1.0920x over previous
import jax
import jax.numpy as jnp
from jax import lax
from jax.experimental import pallas as pl

B, T, D = 64, 1024, 32
K = 1024
TB = 4096


def _vq_body(h_ref, w1_ref, c2_ref, w2_ref, q_ref, idx_ref):
    h = h_ref[...]
    nxc = lax.dot_general(h, w1_ref[...], (((1,), (0,)), ((), ())),
                          preferred_element_type=jnp.float32)
    dist = nxc + c2_ref[...]
    m = jnp.min(dist, axis=1, keepdims=True)
    onehot = (dist <= m).astype(jnp.float32)
    qi = lax.dot_general(onehot, w2_ref[...], (((1,), (0,)), ((), ())),
                         preferred_element_type=jnp.float32)
    q = qi[:, :D]
    q_ref[...] = h + (q - h)
    idx_ref[...] = qi[:, D:D + 1].astype(jnp.int32)


@jax.jit
def kernel(h, codebook):
    bsz, t, d = h.shape
    n = bsz * t
    grid = n // TB
    flat = h.reshape(n, d)
    w1 = -2.0 * codebook.T
    c2 = jnp.sum(codebook ** 2, axis=1)[None, :]
    w2 = jnp.concatenate(
        [codebook, lax.broadcasted_iota(jnp.float32, (K, 1), 0)], axis=1)
    q_flat, idx_col = pl.pallas_call(
        _vq_body,
        grid=(grid,),
        in_specs=[
            pl.BlockSpec((TB, d), lambda i: (i, 0)),
            pl.BlockSpec((d, K), lambda i: (0, 0)),
            pl.BlockSpec((1, K), lambda i: (0, 0)),
            pl.BlockSpec((K, d + 1), lambda i: (0, 0)),
        ],
        out_specs=[
            pl.BlockSpec((TB, d), lambda i: (i, 0)),
            pl.BlockSpec((TB, 1), lambda i: (i, 0)),
        ],
        out_shape=[
            jax.ShapeDtypeStruct((n, d), jnp.float32),
            jax.ShapeDtypeStruct((n, 1), jnp.int32),
        ],
    )(flat, w1, c2, w2)
    return q_flat.reshape(bsz, t, d), idx_col.reshape(bsz, t)


# speedup vs baseline: 1.1026x; 1.0097x over previous
import jax
import jax.numpy as jnp
from jax import lax
from jax.experimental import pallas as pl

B, T, D = 64, 1024, 32
K = 1024
TB = 8192


def _vq_body(h_ref, w1_ref, c2_ref, w2_ref, q_ref, idx_ref):
    h = h_ref[...]
    nxc = lax.dot_general(h, w1_ref[...], (((1,), (0,)), ((), ())),
                          preferred_element_type=jnp.float32)
    dist = nxc + c2_ref[...]
    m = jnp.min(dist, axis=1, keepdims=True)
    onehot = (dist <= m).astype(jnp.float32)
    qi = lax.dot_general(onehot, w2_ref[...], (((1,), (0,)), ((), ())),
                         preferred_element_type=jnp.float32)
    q = qi[:, :D]
    q_ref[...] = h + (q - h)
    idx_ref[...] = qi[:, D:D + 1].astype(jnp.int32)


@jax.jit
def kernel(h, codebook):
    bsz, t, d = h.shape
    n = bsz * t
    grid = n // TB
    flat = h.reshape(n, d)
    w1 = -2.0 * codebook.T
    c2 = jnp.sum(codebook ** 2, axis=1)[None, :]
    w2 = jnp.concatenate(
        [codebook, lax.broadcasted_iota(jnp.float32, (K, 1), 0)], axis=1)
    q_flat, idx_col = pl.pallas_call(
        _vq_body,
        grid=(grid,),
        in_specs=[
            pl.BlockSpec((TB, d), lambda i: (i, 0)),
            pl.BlockSpec((d, K), lambda i: (0, 0)),
            pl.BlockSpec((1, K), lambda i: (0, 0)),
            pl.BlockSpec((K, d + 1), lambda i: (0, 0)),
        ],
        out_specs=[
            pl.BlockSpec((TB, d), lambda i: (i, 0)),
            pl.BlockSpec((TB, 1), lambda i: (i, 0)),
        ],
        out_shape=[
            jax.ShapeDtypeStruct((n, d), jnp.float32),
            jax.ShapeDtypeStruct((n, 1), jnp.int32),
        ],
    )(flat, w1, c2, w2)
    return q_flat.reshape(bsz, t, d), idx_col.reshape(bsz, t)
